# Initial kernel scaffold; baseline (speedup 1.0000x reference)
import jax
import jax.numpy as jnp
from jax.experimental import pallas as pl


def kernel(x, adj, eigvec, eigvalue, Wq, bq, Wk, bk, Wv, bv, Wo, bo, Wp, bp, Wf1, bf1, Wf2, bf2):
    raise NotImplementedError("write your pallas kernel here")


# speedup vs baseline: 2.3802x; 2.3802x over previous
import math

import jax
import jax.numpy as jnp
from jax import lax
from jax.experimental import pallas as pl
from jax.experimental.pallas import tpu as pltpu

_B, _T, _N, _D, _S = 4, 12, 1024, 64, 20
_SAMPLES = int(3 * math.log2(_N))
_PAD_I = 32
_NEG = -3e38
_BIG_I = 2**30
_HI = lax.Precision.HIGHEST
_F32 = jnp.float32
_BF16 = jnp.bfloat16


def _ln_rows(v):
    m = jnp.mean(v, axis=1, keepdims=True)
    c = v - m
    var = jnp.mean(c * c, axis=1, keepdims=True)
    return c / jnp.sqrt(var + 1e-5)


def _bfr(a):
    return a.astype(_BF16).astype(_F32)


def _body(x_r, ev_r, eigval_r, adj_r,
          wq_r, bq_r, wk_r, bk_r, wv_r, bv_r, wo_r, bo_r,
          wp_r, bp_r, wf1_r, bf1_r, wf2_r, bf2_r,
          out_r, g_sc, id_sc):
    N, D = _N, _D

    @pl.when(pl.program_id(0) == 0)
    def _init():
        iota_m = lax.broadcasted_iota(jnp.int32, (N, N), 1)
        acc = jnp.zeros((N, N), _F32)
        for s in range(_S):
            acc = acc + (adj_r[:, s:s + 1] == iota_m).astype(_F32)
        g_sc[...] = acc
        iota_r = lax.broadcasted_iota(jnp.int32, (N, N), 0)
        id_sc[...] = (iota_r == iota_m).astype(_F32)

    p = (ev_r[...].astype(_F32) * eigval_r[...].astype(_F32))
    x_ = x_r[0] + _bfr(p)
    xb = x_.astype(_BF16)
    Q = jnp.dot(xb, wq_r[...], preferred_element_type=_F32) + bq_r[...]
    K = jnp.dot(xb, wk_r[...], preferred_element_type=_F32) + bk_r[...]
    V = jnp.dot(xb, wv_r[...], preferred_element_type=_F32) + bv_r[...]
    Qb, Kb, Vb = Q.astype(_BF16), K.astype(_BF16), V.astype(_BF16)

    C = lax.dot_general(Q, K, (((1,), (1,)), ((), ())),
                        preferred_element_type=_F32, precision=_HI)
    W2 = g_sc[...] * C
    GAT = jnp.dot(W2, V, preferred_element_type=_F32,
                  precision=_HI)
    m_col = jnp.dot(GAT.astype(_BF16), wp_r[...],
                    preferred_element_type=_F32) + bp_r[...]
    m_row = lax.dot_general(m_col, id_sc[...], (((0,), (0,)), ((), ())),
                            preferred_element_type=_F32, precision=_HI)

    iota_l = lax.broadcasted_iota(jnp.int32, (1, N), 1)
    rows = []
    mw = m_row
    for _ in range(_SAMPLES):
        mx = jnp.max(mw, axis=1, keepdims=True)
        idx = jnp.min(jnp.where(mw == mx, iota_l, _BIG_I),
                      axis=1, keepdims=True)
        hit = iota_l == idx
        rows.append(hit.astype(_F32))
        mw = jnp.where(hit, _NEG, mw)
    zero_row = jnp.zeros((1, N), _F32)
    rows.extend([zero_row] * (_PAD_I - _SAMPLES))
    onehot = jnp.concatenate(rows, axis=0)

    q_red = jnp.dot(onehot, Q, preferred_element_type=_F32,
                    precision=_HI)
    qks = lax.dot_general(q_red.astype(_BF16), Kb, (((1,), (1,)), ((), ())),
                          preferred_element_type=_F32) * (1.0 / math.sqrt(D))
    rowmax = jnp.max(qks, axis=1, keepdims=True)
    e = jnp.exp(qks - rowmax)
    z = jnp.sum(e, axis=1, keepdims=True)
    attn = e / z
    val = jnp.dot(attn.astype(_BF16), Vb,
                  preferred_element_type=_F32)

    iota_i = lax.broadcasted_iota(jnp.int32, (_PAD_I, N), 0)
    attn_m = jnp.where(iota_i < _SAMPLES, attn, -1.0)
    colmax = jnp.max(attn_m, axis=0, keepdims=True)
    minidx = jnp.min(jnp.where(attn_m == colmax, iota_i, _BIG_I),
                     axis=0, keepdims=True)
    route = (iota_i == minidx).astype(_F32)
    value = lax.dot_general(route, val, (((0,), (0,)), ((), ())),
                            preferred_element_type=_F32, precision=_HI)

    v1 = (jnp.dot(value.astype(_BF16), wo_r[...],
                  preferred_element_type=_F32) + bo_r[...] + x_)
    v1 = _ln_rows(v1)
    h = jnp.maximum(jnp.dot(v1.astype(_BF16), wf1_r[...],
                            preferred_element_type=_F32) + bf1_r[...], 0.0)
    h = (jnp.dot(h.astype(_BF16), wf2_r[...],
                 preferred_element_type=_F32) + bf2_r[...] + v1)
    out_r[0] = _ln_rows(h)


@jax.jit
def kernel(x, adj, eigvec, eigvalue, Wq, bq, Wk, bk, Wv, bv, Wo, bo,
           Wp, bp, Wf1, bf1, Wf2, bf2):
    BT = _B * _T
    x48 = x.reshape(BT, _N, _D)
    ev = eigvec.reshape(_D, _N).T.astype(_BF16)
    eigval = eigvalue.reshape(1, _D).astype(_BF16)
    row = lambda b: b.reshape(1, -1)
    wb = lambda w: w.astype(_BF16)

    full = lambda shp: pl.BlockSpec(shp, lambda i: (0,) * len(shp))
    out = pl.pallas_call(
        _body,
        grid=(BT,),
        in_specs=[
            pl.BlockSpec((1, _N, _D), lambda i: (i, 0, 0)),
            full((_N, _D)),
            full((1, _D)),
            full((_N, _S)),
            full((_D, _D)), full((1, _D)),
            full((_D, _D)), full((1, _D)),
            full((_D, _D)), full((1, _D)),
            full((_D, _D)), full((1, _D)),
            full((_D, 1)), full((1, 1)),
            full((_D, _D)), full((1, _D)),
            full((_D, _D)), full((1, _D)),
        ],
        out_specs=pl.BlockSpec((1, _N, _D), lambda i: (i, 0, 0)),
        out_shape=jax.ShapeDtypeStruct((BT, _N, _D), jnp.float32),
        scratch_shapes=[
            pltpu.VMEM((_N, _N), jnp.float32),
            pltpu.VMEM((_N, _N), jnp.float32),
        ],
        compiler_params=pltpu.CompilerParams(
            dimension_semantics=("arbitrary",),
        ),
    )(x48, ev, eigval, adj,
      wb(Wq), row(bq), wb(Wk), row(bk), wb(Wv), row(bv), wb(Wo), row(bo),
      wb(Wp), bp.reshape(1, 1), wb(Wf1), row(bf1), wb(Wf2), row(bf2))
    return out.reshape(_B, _T, _N, _D)


# speedup vs baseline: 3.0391x; 1.2768x over previous
import math

import jax
import jax.numpy as jnp
from jax import lax
from jax.experimental import pallas as pl
from jax.experimental.pallas import tpu as pltpu

_B, _T, _N, _D, _S = 4, 12, 1024, 64, 20
_SAMPLES = int(3 * math.log2(_N))
_PAD_I = 32
_SPAD = 24
_NEG = -3e38
_BIG_I = 2**30
_HI = lax.Precision.HIGHEST
_F32 = jnp.float32
_BF16 = jnp.bfloat16


def _ln_rows(v):
    m = jnp.mean(v, axis=1, keepdims=True)
    c = v - m
    var = jnp.mean(c * c, axis=1, keepdims=True)
    return c / jnp.sqrt(var + 1e-5)


def _bfr(a):
    return a.astype(_BF16).astype(_F32)


def _body(x_r, ev_r, eigval_r, adjt_r,
          wq_r, bq_r, wk_r, bk_r, wv_r, bv_r, wo_r, bo_r,
          wp_r, bp_r, wf1_r, bf1_r, wf2_r, bf2_r,
          out_r, gt_sc):
    N, D = _N, _D

    @pl.when(pl.program_id(0) == 0)
    def _init():
        iota_m = lax.broadcasted_iota(jnp.int32, (N, N), 0)
        acc = jnp.zeros((N, N), _F32)
        for s in range(_S):
            acc = acc + (adjt_r[s:s + 1, :] == iota_m).astype(_F32)
        gt_sc[...] = acc

    p = (ev_r[...].astype(_F32) * eigval_r[...].astype(_F32))
    x_ = x_r[0] + _bfr(p)
    xb = x_.astype(_BF16)
    Q = jnp.dot(xb, wq_r[...], preferred_element_type=_F32) + bq_r[...]
    K = jnp.dot(xb, wk_r[...], preferred_element_type=_F32) + bk_r[...]
    V = jnp.dot(xb, wv_r[...], preferred_element_type=_F32) + bv_r[...]
    Kb, Vb = K.astype(_BF16), V.astype(_BF16)

    ct = lax.dot_general(K, Q, (((1,), (1,)), ((), ())),
                         preferred_element_type=_F32, precision=_HI)
    w2t = gt_sc[...] * ct
    gatt = lax.dot_general(V, w2t, (((0,), (0,)), ((), ())),
                           preferred_element_type=_F32, precision=_HI)
    gatb = gatt.astype(_BF16)
    m8 = jnp.concatenate(
        [lax.dot_general(wp_r[...], gatb[:, c * 128:(c + 1) * 128],
                         (((0,), (0,)), ((), ())),
                         preferred_element_type=_F32)
         for c in range(N // 128)], axis=0) + bp_r[...]

    iota_g = (lax.broadcasted_iota(jnp.int32, (8, 128), 0) * 128
              + lax.broadcasted_iota(jnp.int32, (8, 128), 1))
    iota_l = lax.broadcasted_iota(jnp.int32, (1, N), 1)
    rows = []
    mw = m8
    for _ in range(_SAMPLES):
        mx = jnp.max(mw)
        gidx = jnp.min(jnp.where(mw == mx, iota_g, _BIG_I))
        rows.append((iota_l == gidx).astype(_F32))
        mw = jnp.where(iota_g == gidx, _NEG, mw)
    zero_row = jnp.zeros((1, N), _F32)
    rows.extend([zero_row] * (_PAD_I - _SAMPLES))
    onehot = jnp.concatenate(rows, axis=0)

    q_red = jnp.dot(onehot, Q, preferred_element_type=_F32,
                    precision=_HI)
    qks = lax.dot_general(q_red.astype(_BF16), Kb, (((1,), (1,)), ((), ())),
                          preferred_element_type=_F32) * (1.0 / math.sqrt(D))
    rowmax = jnp.max(qks, axis=1, keepdims=True)
    e = jnp.exp(qks - rowmax)
    z = jnp.sum(e, axis=1, keepdims=True)
    attn = e / z
    val = jnp.dot(attn.astype(_BF16), Vb,
                  preferred_element_type=_F32)

    iota_i = lax.broadcasted_iota(jnp.int32, (_PAD_I, N), 0)
    attn_m = jnp.where(iota_i < _SAMPLES, attn, -1.0)
    colmax = jnp.max(attn_m, axis=0, keepdims=True)
    minidx = jnp.min(jnp.where(attn_m == colmax, iota_i, _BIG_I),
                     axis=0, keepdims=True)
    route = (iota_i == minidx).astype(_F32)
    value = lax.dot_general(route, val, (((0,), (0,)), ((), ())),
                            preferred_element_type=_F32, precision=_HI)

    v1 = (jnp.dot(value.astype(_BF16), wo_r[...],
                  preferred_element_type=_F32) + bo_r[...] + x_)
    v1 = _ln_rows(v1)
    h = jnp.maximum(jnp.dot(v1.astype(_BF16), wf1_r[...],
                            preferred_element_type=_F32) + bf1_r[...], 0.0)
    h = (jnp.dot(h.astype(_BF16), wf2_r[...],
                 preferred_element_type=_F32) + bf2_r[...] + v1)
    out_r[0] = _ln_rows(h)


@jax.jit
def kernel(x, adj, eigvec, eigvalue, Wq, bq, Wk, bk, Wv, bv, Wo, bo,
           Wp, bp, Wf1, bf1, Wf2, bf2):
    BT = _B * _T
    x48 = x.reshape(BT, _N, _D)
    adjt = jnp.pad(adj.T, ((0, _SPAD - _S), (0, 0)))
    ev = eigvec.reshape(_D, _N).T.astype(_BF16)
    eigval = eigvalue.reshape(1, _D).astype(_BF16)
    row = lambda b: b.reshape(1, -1)
    wb = lambda w: w.astype(_BF16)

    full = lambda shp: pl.BlockSpec(shp, lambda i: (0,) * len(shp))
    out = pl.pallas_call(
        _body,
        grid=(BT,),
        in_specs=[
            pl.BlockSpec((1, _N, _D), lambda i: (i, 0, 0)),
            full((_N, _D)),
            full((1, _D)),
            full((_SPAD, _N)),
            full((_D, _D)), full((1, _D)),
            full((_D, _D)), full((1, _D)),
            full((_D, _D)), full((1, _D)),
            full((_D, _D)), full((1, _D)),
            full((_D, 1)), full((1, 1)),
            full((_D, _D)), full((1, _D)),
            full((_D, _D)), full((1, _D)),
        ],
        out_specs=pl.BlockSpec((1, _N, _D), lambda i: (i, 0, 0)),
        out_shape=jax.ShapeDtypeStruct((BT, _N, _D), jnp.float32),
        scratch_shapes=[
            pltpu.VMEM((_N, _N), jnp.float32),
        ],
        compiler_params=pltpu.CompilerParams(
            dimension_semantics=("arbitrary",),
        ),
    )(x48, ev, eigval, adjt,
      wb(Wq), row(bq), wb(Wk), row(bk), wb(Wv), row(bv), wb(Wo), row(bo),
      wb(Wp), bp.reshape(1, 1), wb(Wf1), row(bf1), wb(Wf2), row(bf2))
    return out.reshape(_B, _T, _N, _D)


# speedup vs baseline: 3.3836x; 1.1134x over previous
import math

import jax
import jax.numpy as jnp
from jax import lax
from jax.experimental import pallas as pl
from jax.experimental.pallas import tpu as pltpu

_B, _T, _N, _D, _S = 4, 12, 1024, 64, 20
_SAMPLES = int(3 * math.log2(_N))
_PAD_I = 32
_SPAD = 24
_NEG = -3e38
_BIG_I = 2**30
_SL = 2
_HI = lax.Precision.HIGHEST
_F32 = jnp.float32
_BF16 = jnp.bfloat16


def _ln_rows(v):
    m = jnp.mean(v, axis=1, keepdims=True)
    c = v - m
    var = jnp.mean(c * c, axis=1, keepdims=True)
    return c / jnp.sqrt(var + 1e-5)


def _bfr(a):
    return a.astype(_BF16).astype(_F32)


def _body(x_r, ev_r, eigval_r, adjt_r,
          wq_r, bq_r, wk_r, bk_r, wv_r, bv_r, wo_r, bo_r,
          wp_r, bp_r, wf1_r, bf1_r, wf2_r, bf2_r,
          out_r, gt_sc):
    N, D = _N, _D

    @pl.when(pl.program_id(0) == 0)
    def _init():
        iota_m = lax.broadcasted_iota(jnp.int32, (N, N), 0)
        acc = jnp.zeros((N, N), _F32)
        for s in range(_S):
            acc = acc + (adjt_r[s:s + 1, :] == iota_m).astype(_F32)
        gt_sc[...] = acc

    for j in range(_SL):
        _slice_body(x_r, ev_r, eigval_r,
                    wq_r, bq_r, wk_r, bk_r, wv_r, bv_r, wo_r, bo_r,
                    wp_r, bp_r, wf1_r, bf1_r, wf2_r, bf2_r,
                    out_r, gt_sc, j)


def _slice_body(x_r, ev_r, eigval_r,
                wq_r, bq_r, wk_r, bk_r, wv_r, bv_r, wo_r, bo_r,
                wp_r, bp_r, wf1_r, bf1_r, wf2_r, bf2_r,
                out_r, gt_sc, j):
    N, D = _N, _D
    p = (ev_r[...].astype(_F32) * eigval_r[...].astype(_F32))
    x_ = x_r[j] + _bfr(p)
    xb = x_.astype(_BF16)
    Q = jnp.dot(xb, wq_r[...], preferred_element_type=_F32) + bq_r[...]
    K = jnp.dot(xb, wk_r[...], preferred_element_type=_F32) + bk_r[...]
    V = jnp.dot(xb, wv_r[...], preferred_element_type=_F32) + bv_r[...]
    Kb, Vb = K.astype(_BF16), V.astype(_BF16)

    ct = lax.dot_general(K, Q, (((1,), (1,)), ((), ())),
                         preferred_element_type=_F32, precision=_HI)
    w2t = gt_sc[...] * ct
    gatt = lax.dot_general(V, w2t, (((0,), (0,)), ((), ())),
                           preferred_element_type=_F32, precision=_HI)
    gatb = gatt.astype(_BF16)
    m8 = jnp.concatenate(
        [lax.dot_general(wp_r[...], gatb[:, c * 128:(c + 1) * 128],
                         (((0,), (0,)), ((), ())),
                         preferred_element_type=_F32)
         for c in range(N // 128)], axis=0) + bp_r[...]

    iota_g = (lax.broadcasted_iota(jnp.int32, (8, 128), 0) * 128
              + lax.broadcasted_iota(jnp.int32, (8, 128), 1))
    iota_l = lax.broadcasted_iota(jnp.int32, (1, N), 1)
    gids = []
    mw = m8
    for _ in range(_SAMPLES):
        mx = jnp.max(jnp.max(mw, axis=1, keepdims=True), axis=0,
                     keepdims=True)
        sel = jnp.where(mw == mx, iota_g, _BIG_I)
        gidx = jnp.min(jnp.min(sel, axis=1, keepdims=True), axis=0,
                       keepdims=True)
        gids.append(gidx)
        mw = jnp.where(iota_g == gidx, _NEG, mw)
    gids.extend([jnp.full((1, 1), -1, jnp.int32)] * (_PAD_I - _SAMPLES))
    gcol = jnp.concatenate(gids, axis=0)
    onehot = (gcol == iota_l).astype(_F32)

    q_red = jnp.dot(onehot, Q, preferred_element_type=_F32,
                    precision=_HI)
    qks = lax.dot_general(q_red.astype(_BF16), Kb, (((1,), (1,)), ((), ())),
                          preferred_element_type=_F32) * (1.0 / math.sqrt(D))
    rowmax = jnp.max(qks, axis=1, keepdims=True)
    e = jnp.exp(qks - rowmax)
    z = jnp.sum(e, axis=1, keepdims=True)
    attn = e / z
    val = jnp.dot(attn.astype(_BF16), Vb,
                  preferred_element_type=_F32)

    iota_i = lax.broadcasted_iota(jnp.int32, (_PAD_I, N), 0)
    attn_m = jnp.where(iota_i < _SAMPLES, attn, -1.0)
    colmax = jnp.max(attn_m, axis=0, keepdims=True)
    minidx = jnp.min(jnp.where(attn_m == colmax, iota_i, _BIG_I),
                     axis=0, keepdims=True)
    route = (iota_i == minidx).astype(_F32)
    value = lax.dot_general(route, val, (((0,), (0,)), ((), ())),
                            preferred_element_type=_F32, precision=_HI)

    v1 = (jnp.dot(value.astype(_BF16), wo_r[...],
                  preferred_element_type=_F32) + bo_r[...] + x_)
    v1 = _ln_rows(v1)
    h = jnp.maximum(jnp.dot(v1.astype(_BF16), wf1_r[...],
                            preferred_element_type=_F32) + bf1_r[...], 0.0)
    h = (jnp.dot(h.astype(_BF16), wf2_r[...],
                 preferred_element_type=_F32) + bf2_r[...] + v1)
    out_r[j] = _ln_rows(h)


@jax.jit
def kernel(x, adj, eigvec, eigvalue, Wq, bq, Wk, bk, Wv, bv, Wo, bo,
           Wp, bp, Wf1, bf1, Wf2, bf2):
    BT = _B * _T
    x48 = x.reshape(BT, _N, _D)
    adjt = jnp.pad(adj.T, ((0, _SPAD - _S), (0, 0)))
    ev = eigvec.reshape(_D, _N).T.astype(_BF16)
    eigval = eigvalue.reshape(1, _D).astype(_BF16)
    row = lambda b: b.reshape(1, -1)
    wb = lambda w: w.astype(_BF16)

    full = lambda shp: pl.BlockSpec(shp, lambda i: (0,) * len(shp))
    out = pl.pallas_call(
        _body,
        grid=(BT // _SL,),
        in_specs=[
            pl.BlockSpec((_SL, _N, _D), lambda i: (i, 0, 0)),
            full((_N, _D)),
            full((1, _D)),
            full((_SPAD, _N)),
            full((_D, _D)), full((1, _D)),
            full((_D, _D)), full((1, _D)),
            full((_D, _D)), full((1, _D)),
            full((_D, _D)), full((1, _D)),
            full((_D, 1)), full((1, 1)),
            full((_D, _D)), full((1, _D)),
            full((_D, _D)), full((1, _D)),
        ],
        out_specs=pl.BlockSpec((_SL, _N, _D), lambda i: (i, 0, 0)),
        out_shape=jax.ShapeDtypeStruct((BT, _N, _D), jnp.float32),
        scratch_shapes=[
            pltpu.VMEM((_N, _N), jnp.float32),
        ],
        compiler_params=pltpu.CompilerParams(
            dimension_semantics=("arbitrary",),
        ),
    )(x48, ev, eigval, adjt,
      wb(Wq), row(bq), wb(Wk), row(bk), wb(Wv), row(bv), wb(Wo), row(bo),
      wb(Wp), bp.reshape(1, 1), wb(Wf1), row(bf1), wb(Wf2), row(bf2))
    return out.reshape(_B, _T, _N, _D)


# speedup vs baseline: 4.0161x; 1.1869x over previous
import math

import jax
import jax.numpy as jnp
from jax import lax
from jax.experimental import pallas as pl
from jax.experimental.pallas import tpu as pltpu

_B, _T, _N, _D, _S = 4, 12, 1024, 64, 20
_SAMPLES = int(3 * math.log2(_N))
_PAD_I = 32
_SPAD = 24
_NEG = -3e38
_BIG_I = 2**30
_SL = 4
_HI = lax.Precision.HIGHEST
_F32 = jnp.float32
_BF16 = jnp.bfloat16


def _ln_rows(v):
    m = jnp.mean(v, axis=1, keepdims=True)
    c = v - m
    var = jnp.mean(c * c, axis=1, keepdims=True)
    return c * lax.rsqrt(var + 1e-5)


def _bfr(a):
    return a.astype(_BF16).astype(_F32)


def _body(x_r, ev_r, eigval_r, adjt_r,
          wq_r, bq_r, wk_r, bk_r, wv_r, bv_r, wo_r, bo_r,
          wp_r, bp_r, wf1_r, bf1_r, wf2_r, bf2_r,
          out_r, gt_sc):
    N, D = _N, _D

    @pl.when(pl.program_id(0) == 0)
    def _init():
        iota_m = lax.broadcasted_iota(jnp.int32, (N, N), 0)
        acc = jnp.zeros((N, N), _F32)
        for s in range(_S):
            acc = acc + (adjt_r[s:s + 1, :] == iota_m).astype(_F32)
        gt_sc[...] = acc

    sl = []
    for j in range(_SL):
        p = (ev_r[...].astype(_F32) * eigval_r[...].astype(_F32))
        x_ = x_r[j] + _bfr(p)
        xb = x_.astype(_BF16)
        Q = jnp.dot(xb, wq_r[...], preferred_element_type=_F32) + bq_r[...]
        K = jnp.dot(xb, wk_r[...], preferred_element_type=_F32) + bk_r[...]
        V = jnp.dot(xb, wv_r[...], preferred_element_type=_F32) + bv_r[...]

        ct = lax.dot_general(K, Q, (((1,), (1,)), ((), ())),
                             preferred_element_type=_F32, precision=_HI)
        w2t = gt_sc[...] * ct
        gatt = lax.dot_general(V, w2t, (((0,), (0,)), ((), ())),
                               preferred_element_type=_F32, precision=_HI)
        gatb = gatt.astype(_BF16)
        m8 = jnp.concatenate(
            [lax.dot_general(wp_r[...], gatb[:, c * 128:(c + 1) * 128],
                             (((0,), (0,)), ((), ())),
                             preferred_element_type=_F32)
             for c in range(N // 128)], axis=0) + bp_r[...]
        sl.append({"x_": x_, "Q": Q, "Kb": K.astype(_BF16),
                   "Vb": V.astype(_BF16), "m8": m8})

    iota_g = (lax.broadcasted_iota(jnp.int32, (8, 128), 0) * 128
              + lax.broadcasted_iota(jnp.int32, (8, 128), 1))
    iota_l = lax.broadcasted_iota(jnp.int32, (1, N), 1)
    mws = [s["m8"] for s in sl]
    gids = [[] for _ in range(_SL)]
    for _ in range(_SAMPLES):
        for j in range(_SL):
            mw = mws[j]
            mx = jnp.max(jnp.max(mw, axis=1, keepdims=True), axis=0,
                         keepdims=True)
            hit = mw == mx
            gidx = jnp.min(jnp.min(jnp.where(hit, iota_g, _BIG_I),
                                   axis=1, keepdims=True),
                           axis=0, keepdims=True)
            gids[j].append(gidx)
            mws[j] = jnp.where(hit, _NEG, mw)
    pad = [jnp.full((1, 1), -1, jnp.int32)] * (_PAD_I - _SAMPLES)
    onehots = [(jnp.concatenate(g + pad, axis=0) == iota_l).astype(_F32)
               for g in gids]

    iota_i = lax.broadcasted_iota(jnp.int32, (_PAD_I, N), 0)
    for j in range(_SL):
        s = sl[j]
        x_, Q, Kb, Vb = s["x_"], s["Q"], s["Kb"], s["Vb"]
        q_red = jnp.dot(onehots[j], Q, preferred_element_type=_F32,
                        precision=_HI)
        qks = lax.dot_general(q_red.astype(_BF16), Kb,
                              (((1,), (1,)), ((), ())),
                              preferred_element_type=_F32) * (1.0 / math.sqrt(D))
        rowmax = jnp.max(qks, axis=1, keepdims=True)
        e = jnp.exp(qks - rowmax)
        z = jnp.sum(e, axis=1, keepdims=True)
        attn = e / z
        val = jnp.dot(attn.astype(_BF16), Vb,
                      preferred_element_type=_F32)

        attn_m = jnp.where(iota_i < _SAMPLES, attn, -1.0)
        colmax = jnp.max(attn_m, axis=0, keepdims=True)
        minidx = jnp.min(jnp.where(attn_m == colmax, iota_i, _BIG_I),
                         axis=0, keepdims=True)
        route = (iota_i == minidx).astype(_F32)
        value = lax.dot_general(route, val, (((0,), (0,)), ((), ())),
                                preferred_element_type=_F32, precision=_HI)

        v1 = (jnp.dot(value.astype(_BF16), wo_r[...],
                      preferred_element_type=_F32) + bo_r[...] + x_)
        v1 = _ln_rows(v1)
        h = jnp.maximum(jnp.dot(v1.astype(_BF16), wf1_r[...],
                                preferred_element_type=_F32) + bf1_r[...],
                        0.0)
        h = (jnp.dot(h.astype(_BF16), wf2_r[...],
                     preferred_element_type=_F32) + bf2_r[...] + v1)
        out_r[j] = _ln_rows(h)


@jax.jit
def kernel(x, adj, eigvec, eigvalue, Wq, bq, Wk, bk, Wv, bv, Wo, bo,
           Wp, bp, Wf1, bf1, Wf2, bf2):
    BT = _B * _T
    x48 = x.reshape(BT, _N, _D)
    adjt = jnp.pad(adj.T, ((0, _SPAD - _S), (0, 0)))
    ev = eigvec.reshape(_D, _N).T.astype(_BF16)
    eigval = eigvalue.reshape(1, _D).astype(_BF16)
    row = lambda b: b.reshape(1, -1)
    wb = lambda w: w.astype(_BF16)

    full = lambda shp: pl.BlockSpec(shp, lambda i: (0,) * len(shp))
    out = pl.pallas_call(
        _body,
        grid=(BT // _SL,),
        in_specs=[
            pl.BlockSpec((_SL, _N, _D), lambda i: (i, 0, 0)),
            full((_N, _D)),
            full((1, _D)),
            full((_SPAD, _N)),
            full((_D, _D)), full((1, _D)),
            full((_D, _D)), full((1, _D)),
            full((_D, _D)), full((1, _D)),
            full((_D, _D)), full((1, _D)),
            full((_D, 1)), full((1, 1)),
            full((_D, _D)), full((1, _D)),
            full((_D, _D)), full((1, _D)),
        ],
        out_specs=pl.BlockSpec((_SL, _N, _D), lambda i: (i, 0, 0)),
        out_shape=jax.ShapeDtypeStruct((BT, _N, _D), jnp.float32),
        scratch_shapes=[
            pltpu.VMEM((_N, _N), jnp.float32),
        ],
        compiler_params=pltpu.CompilerParams(
            dimension_semantics=("arbitrary",),
        ),
    )(x48, ev, eigval, adjt,
      wb(Wq), row(bq), wb(Wk), row(bk), wb(Wv), row(bv), wb(Wo), row(bo),
      wb(Wp), bp.reshape(1, 1), wb(Wf1), row(bf1), wb(Wf2), row(bf2))
    return out.reshape(_B, _T, _N, _D)
